# baseline (speedup 1.0000x reference)
import functools

import jax
import jax.numpy as jnp
from jax import lax
from jax.experimental import pallas as pl
from jax.experimental.pallas import tpu as pltpu
from jax.experimental.pallas import tpu_sc as plsc

VOCAB = 32128
D_MODEL = 768
BATCH = 4
SEQ = 2048
TOTAL = BATCH * SEQ

NUM_CORES = 2
NUM_SUBCORES = 16
NUM_WORKERS = NUM_CORES * NUM_SUBCORES
PER_WORKER = TOTAL // NUM_WORKERS
CHUNK = 64
NUM_CHUNKS = PER_WORKER // CHUNK

_mesh = plsc.VectorSubcoreMesh(core_axis_name="c", subcore_axis_name="s")


@functools.partial(
    pl.kernel,
    mesh=_mesh,
    out_type=jax.ShapeDtypeStruct((TOTAL, D_MODEL), jnp.float32),
    scratch_types=[
        pltpu.VMEM((NUM_CHUNKS, CHUNK), jnp.int32),
        pltpu.VMEM((CHUNK, D_MODEL), jnp.float32),
        pltpu.VMEM((CHUNK, D_MODEL), jnp.float32),
        pltpu.SemaphoreType.DMA,
        pltpu.SemaphoreType.DMA,
        pltpu.SemaphoreType.DMA,
        pltpu.SemaphoreType.DMA,
    ],
)
def _embed_gather(x_hbm, w_hbm, out_hbm, idx_v, rows0, rows1,
                  gsem0, gsem1, ssem0, ssem1):
    wid = lax.axis_index("s") * NUM_CORES + lax.axis_index("c")
    base = wid * PER_WORKER
    rows = (rows0, rows1)
    gsems = (gsem0, gsem1)
    ssems = (ssem0, ssem1)

    pltpu.sync_copy(x_hbm.at[wid], idx_v)

    gathers = [None, None]
    scatters = [None, None]
    gathers[0] = pltpu.async_copy(w_hbm.at[idx_v.at[0]], rows[0], gsems[0])
    for c in range(NUM_CHUNKS):
        slot = c % 2
        nslot = (c + 1) % 2
        if c + 1 < NUM_CHUNKS:
            if scatters[nslot] is not None:
                scatters[nslot].wait()
            gathers[nslot] = pltpu.async_copy(
                w_hbm.at[idx_v.at[c + 1]], rows[nslot], gsems[nslot])
        gathers[slot].wait()
        scatters[slot] = pltpu.async_copy(
            rows[slot], out_hbm.at[pl.ds(base + c * CHUNK, CHUNK)],
            ssems[slot])
    scatters[0].wait()
    scatters[1].wait()


def kernel(x, embed_weight):
    idx = x.reshape(NUM_WORKERS, NUM_CHUNKS, CHUNK).astype(jnp.int32)
    out = _embed_gather(idx, embed_weight)
    return out.reshape(BATCH, SEQ, D_MODEL)


# speedup vs baseline: 1.0081x; 1.0081x over previous
import functools

import jax
import jax.numpy as jnp
from jax import lax
from jax.experimental import pallas as pl
from jax.experimental.pallas import tpu as pltpu
from jax.experimental.pallas import tpu_sc as plsc

VOCAB = 32128
D_MODEL = 768
BATCH = 4
SEQ = 2048
TOTAL = BATCH * SEQ

NUM_CORES = 2
NUM_SUBCORES = 16
NUM_WORKERS = NUM_CORES * NUM_SUBCORES
PER_WORKER = TOTAL // NUM_WORKERS
W_PER_ROW = SEQ // PER_WORKER
CHUNK = 128
NUM_CHUNKS = PER_WORKER // CHUNK

_mesh = plsc.VectorSubcoreMesh(core_axis_name="c", subcore_axis_name="s")


@functools.partial(
    pl.kernel,
    mesh=_mesh,
    out_type=jax.ShapeDtypeStruct((TOTAL, D_MODEL), jnp.float32),
    scratch_types=[
        pltpu.VMEM((CHUNK,), jnp.int32),
        pltpu.VMEM((CHUNK, D_MODEL), jnp.float32),
        pltpu.SemaphoreType.DMA,
    ],
)
def _embed_gather(x_hbm, w_hbm, out_hbm, idx_v, rows_v, sem):
    wid = lax.axis_index("s") * NUM_CORES + lax.axis_index("c")
    b = wid // W_PER_ROW
    col = (wid % W_PER_ROW) * PER_WORKER
    base = wid * PER_WORKER
    for c in range(NUM_CHUNKS):
        pltpu.sync_copy(x_hbm.at[b, pl.ds(col + c * CHUNK, CHUNK)], idx_v)
        pltpu.async_copy(w_hbm.at[idx_v], rows_v, sem).wait()
        pltpu.sync_copy(rows_v, out_hbm.at[pl.ds(base + c * CHUNK, CHUNK)])


def kernel(x, embed_weight):
    out = _embed_gather(x.astype(jnp.int32), embed_weight)
    return out.reshape(BATCH, SEQ, D_MODEL)
